# Initial kernel scaffold; baseline (speedup 1.0000x reference)
import jax
import jax.numpy as jnp
from jax.experimental import pallas as pl


def kernel(x, t, history, W_hist):
    raise NotImplementedError("write your pallas kernel here")


# speedup vs baseline: 1.4746x; 1.4746x over previous
import functools

import jax
import jax.numpy as jnp
from jax.experimental import pallas as pl
from jax.experimental.pallas import tpu as pltpu

QT = 256
HC = 512


def _tc_body(t_ref, h_ref, x_ref, w_ref, out_ref, m1_s, m2_s):
    j = pl.program_id(1)
    nj = pl.num_programs(1)

    @pl.when(j == 0)
    def _init():
        m1_s[...] = jnp.full((QT, HC), -jnp.inf, jnp.float32)
        m2_s[...] = jnp.full((QT, HC), -jnp.inf, jnp.float32)

    tb = t_ref[...]
    hseg = h_ref[...]
    hb = jnp.broadcast_to(hseg, (QT, HC))
    masked = jnp.where(hb <= tb, hb, -jnp.inf)
    o1 = m1_s[...]
    m1_s[...] = jnp.maximum(o1, masked)
    m2_s[...] = jnp.maximum(m2_s[...], jnp.minimum(o1, masked))

    @pl.when(j == nj - 1)
    def _finalize():
        m1 = m1_s[...]
        m2 = m2_s[...]
        m1r = jnp.max(m1, axis=1, keepdims=True)
        eq = m1 == m1r
        lane = jax.lax.broadcasted_iota(jnp.int32, (QT, HC), 1)
        first_idx = jnp.min(jnp.where(eq, lane, HC), axis=1, keepdims=True)
        first = lane == first_idx
        sm = jnp.max(jnp.where(first, -jnp.inf, m1), axis=1)
        m2c = jnp.max(jnp.where(first, m2, -jnp.inf), axis=1)
        m2r = jnp.maximum(sm, m2c)
        tq = tb[:, 0]
        e1 = jnp.exp(m1r[:, 0] - tq)
        e2 = jnp.exp(m2r - tq)
        w0 = w_ref[0, 0]
        w1 = w_ref[0, 1]
        hv = w0 * e1 + w1 * e2
        out_ref[...] = hv[None, :] - x_ref[...]


@jax.jit
def kernel(x, t, history, W_hist):
    B, S = x.shape
    H = history.shape[0]
    t_row = t[0]
    hist2d = history.reshape(1, H)
    grid = (S // QT, H // HC)
    out = pl.pallas_call(
        _tc_body,
        grid=grid,
        in_specs=[
            pl.BlockSpec((QT, 1), lambda i, j: (i, 0)),
            pl.BlockSpec((1, HC), lambda i, j: (0, j)),
            pl.BlockSpec((B, QT), lambda i, j: (0, i)),
            pl.BlockSpec((1, 2), lambda i, j: (0, 0)),
        ],
        out_specs=pl.BlockSpec((B, QT), lambda i, j: (0, i)),
        out_shape=jax.ShapeDtypeStruct((B, S), jnp.float32),
        scratch_shapes=[
            pltpu.VMEM((QT, HC), jnp.float32),
            pltpu.VMEM((QT, HC), jnp.float32),
        ],
    )(t_row, hist2d, x, W_hist)
    return out


# speedup vs baseline: 3.1271x; 2.1207x over previous
import functools

import jax
import jax.numpy as jnp
from jax import lax
from jax.experimental import pallas as pl
from jax.experimental.pallas import tpu as pltpu
from jax.experimental.pallas import tpu_sc as plsc

NW = 32
L = 16
NC = 2
H = 32768
S = 2048


def _sc_body(hist_hbm, t_hbm, w0_hbm, w1_hbm, hv_hbm,
             hist_v, t_v, w0_v, w1_v, list_v, qval_v, qidx_v, hvout_v, sem):
    c = lax.axis_index("c")
    s = lax.axis_index("s")
    w = (s * NC + c).astype(jnp.int32)

    pltpu.sync_copy(hist_hbm, hist_v)
    pltpu.sync_copy(t_hbm, t_v)
    pltpu.sync_copy(w0_hbm, w0_v)
    pltpu.sync_copy(w1_hbm, w1_v)

    iota = lax.broadcasted_iota(jnp.int32, (L,), 0)
    ninf = jnp.full((L,), -jnp.inf, jnp.float32)
    zero_i = jnp.zeros((L,), jnp.int32)
    nw1_i = jnp.full((L,), NW - 1, jnp.int32)
    nw_f = jnp.full((L,), float(NW), jnp.float32)
    one_i = jnp.full((L,), 1, jnp.int32)
    lsac_i = jnp.full((L,), H + L - 1, jnp.int32)
    qsac_i = jnp.full((L,), S + L - 1, jnp.int32)
    big_i = jnp.full((L,), L, jnp.int32)
    wv = jnp.broadcast_to(w, (L,))

    def bcast(a):
        return jnp.broadcast_to(a, (L,))

    def p1_body(i, carry):
        off, p1, p2 = carry
        v = hist_v[pl.ds(i * L, L)]
        b = jnp.minimum(jnp.maximum((v * nw_f).astype(jnp.int32), zero_i), nw1_i)
        bv = jnp.where(b < wv, v, ninf)
        n1 = jnp.maximum(p1, bv)
        p2 = jnp.maximum(p2, jnp.minimum(p1, bv))
        inr = b == wv
        cum = plsc.cumsum(inr.astype(jnp.int32))
        pos = jnp.where(inr, bcast(off) + cum - one_i, lsac_i)
        plsc.store_scatter(list_v, [pos], v)
        return off + jnp.sum(inr.astype(jnp.int32)), n1, p2

    off, p1, p2 = lax.fori_loop(
        0, H // L, p1_body, (jnp.int32(0), ninf, ninf))

    def p2_body(i, qoff):
        tv = t_v[pl.ds(i * L, L)]
        qb = jnp.minimum(jnp.maximum((tv * nw_f).astype(jnp.int32), zero_i), nw1_i)
        qm = qb == wv
        cum = plsc.cumsum(qm.astype(jnp.int32))
        pos = jnp.where(qm, bcast(qoff) + cum - one_i, qsac_i)
        plsc.store_scatter(qval_v, [pos], tv)
        plsc.store_scatter(qidx_v, [pos], iota + bcast(i * L))
        return qoff + jnp.sum(qm.astype(jnp.int32))

    qcnt = lax.fori_loop(0, S // L, p2_body, jnp.int32(0))

    nc_chunks = (off + L - 1) // L
    w0v = w0_v[...]
    w1v = w1_v[...]
    offv = bcast(off)

    def q_body(q, carry):
        tqv = plsc.load_gather(qval_v, [bcast(q)])

        def c_body(j, ac):
            a1, a2 = ac
            lv = list_v[pl.ds(j * L, L)]
            ok = ((iota + bcast(j * L)) < offv) & (lv <= tqv)
            g = jnp.where(ok, lv, ninf)
            n1 = jnp.maximum(a1, g)
            a2 = jnp.maximum(a2, jnp.minimum(a1, g))
            return n1, a2

        a1, a2 = lax.fori_loop(0, nc_chunks, c_body, (ninf, ninf))
        c1 = jnp.maximum(a1, p1)
        c2 = jnp.maximum(jnp.minimum(a1, p1), jnp.maximum(a2, p2))
        m1v = bcast(jnp.max(c1))
        fiv = bcast(jnp.min(jnp.where(c1 == m1v, iota, big_i)))
        first = iota == fiv
        sm = jnp.max(jnp.where(first, ninf, c1))
        m2c = jnp.max(jnp.where(first, c2, ninf))
        m2v = bcast(jnp.maximum(sm, m2c))
        hvv = w0v * jnp.exp(m1v - tqv) + w1v * jnp.exp(m2v - tqv)
        plsc.store_scatter(hvout_v, [jnp.where(iota == zero_i, bcast(q), qsac_i)],
                           hvv)
        return carry

    lax.fori_loop(0, qcnt, q_body, jnp.int32(0))

    qcntv = bcast(qcnt)

    def s_body(j, carry):
        idxv = qidx_v[pl.ds(j * L, L)]
        valid = (iota + bcast(j * L)) < qcntv
        idxv = jnp.where(valid, idxv, jnp.full((L,), S, jnp.int32))
        pltpu.async_copy(hvout_v.at[pl.ds(j * L, L)], hv_hbm.at[idxv],
                         sem).wait()
        return carry

    lax.fori_loop(0, (qcnt + L - 1) // L, s_body, jnp.int32(0))


_sc_kernel = functools.partial(
    pl.kernel,
    out_type=jax.ShapeDtypeStruct((S + L,), jnp.float32),
    mesh=plsc.VectorSubcoreMesh(core_axis_name="c", subcore_axis_name="s"),
    scratch_types=[
        pltpu.VMEM((H,), jnp.float32),
        pltpu.VMEM((S,), jnp.float32),
        pltpu.VMEM((L,), jnp.float32),
        pltpu.VMEM((L,), jnp.float32),
        pltpu.VMEM((H + L,), jnp.float32),
        pltpu.VMEM((S + L,), jnp.float32),
        pltpu.VMEM((S + L,), jnp.int32),
        pltpu.VMEM((S + L,), jnp.float32),
        pltpu.SemaphoreType.DMA,
    ],
    compiler_params=pltpu.CompilerParams(needs_layout_passes=False),
)(_sc_body)


def _combine_body(hv_ref, x_ref, out_ref):
    out_ref[...] = hv_ref[...] - x_ref[...]


@jax.jit
def kernel(x, t, history, W_hist):
    B = x.shape[0]
    t_row = t[0, :, 0]
    w0 = jnp.full((L,), W_hist[0, 0], jnp.float32)
    w1 = jnp.full((L,), W_hist[0, 1], jnp.float32)
    hv = _sc_kernel(history, t_row, w0, w1)
    hv2d = hv[:S].reshape(1, S)
    out = pl.pallas_call(
        _combine_body,
        out_shape=jax.ShapeDtypeStruct((B, S), jnp.float32),
    )(hv2d, x)
    return out


# speedup vs baseline: 3.3281x; 1.0643x over previous
import functools

import jax
import jax.numpy as jnp
from jax import lax
from jax.experimental import pallas as pl
from jax.experimental.pallas import tpu as pltpu
from jax.experimental.pallas import tpu_sc as plsc

NW = 32
L = 16
NC = 2
SB = 8
H = 32768
S = 2048
LCAP = H + L


def _sc_body(hist_hbm, t_hbm, w0_hbm, w1_hbm, hv_hbm,
             hist_v, t_v, w0_v, w1_v, list_v, list2_v, qval_v, qidx_v,
             hvout_v, sbstart_v, sbend_v, sem):
    c = lax.axis_index("c")
    s = lax.axis_index("s")
    w = (s * NC + c).astype(jnp.int32)

    pltpu.sync_copy(hist_hbm, hist_v)
    pltpu.sync_copy(t_hbm, t_v)
    pltpu.sync_copy(w0_hbm, w0_v)
    pltpu.sync_copy(w1_hbm, w1_v)

    iota = lax.broadcasted_iota(jnp.int32, (L,), 0)
    ninf = jnp.full((L,), -jnp.inf, jnp.float32)
    pinf = jnp.full((L,), jnp.inf, jnp.float32)
    one_i = jnp.full((L,), 1, jnp.int32)
    lsac_i = jnp.full((L,), LCAP - 1, jnp.int32)
    qsac_i = jnp.full((L,), S + L - 1, jnp.int32)
    big_i = jnp.full((L,), L, jnp.int32)
    zero_i = jnp.zeros((L,), jnp.int32)
    sb1_i = jnp.full((L,), SB - 1, jnp.int32)

    def bcast(a):
        return jnp.broadcast_to(a, (L,))

    wf = w.astype(jnp.float32)
    lo_f = wf * (1.0 / NW)
    lo_fv = bcast(lo_f)
    lov = jnp.where(w == 0, -jnp.inf, lo_f)
    hiv = jnp.where(w == NW - 1, jnp.inf, (wf + 1.0) * (1.0 / NW))
    lovv = bcast(lov)
    hivv = bcast(hiv)
    sb_scale = jnp.full((L,), float(NW * SB), jnp.float32)

    def sub_bucket(vals):
        r = ((vals - lo_fv) * sb_scale).astype(jnp.int32)
        return jnp.minimum(jnp.maximum(r, zero_i), sb1_i)

    def p1_body(i, carry):
        off, p1, p2 = carry
        v = hist_v[pl.ds(i * L, L)]
        below = v < lovv
        bv = jnp.where(below, v, ninf)
        n1 = jnp.maximum(p1, bv)
        p2 = jnp.maximum(p2, jnp.minimum(p1, bv))
        inr = (v >= lovv) & (v < hivv)
        cum = plsc.cumsum(inr.astype(jnp.int32))
        pos = jnp.where(inr, bcast(off) + cum - one_i, lsac_i)
        plsc.store_scatter(list_v, [pos], v)
        return off + jnp.sum(inr.astype(jnp.int32)), n1, p2

    off, p1, p2 = lax.fori_loop(
        0, H // L, p1_body, (jnp.int32(0), ninf, ninf))
    list_v[pl.ds(off, L)] = ninf
    nc_chunks = (off + L - 1) // L

    subm1 = ninf
    subm2 = ninf
    startv = zero_i
    endv = zero_i
    rtop = jnp.int32(0)
    for sb in range(SB):
        sbv = jnp.full((L,), sb, jnp.int32)
        lane_is_sb = iota == sbv
        startv = jnp.where(lane_is_sb, bcast(rtop), startv)

        def rp_body(j, carry, sbv=sbv):
            rt, a1, a2 = carry
            lv = list_v[pl.ds(j * L, L)]
            m = sub_bucket(lv) == sbv
            g = jnp.where(m, lv, ninf)
            n1 = jnp.maximum(a1, g)
            a2 = jnp.maximum(a2, jnp.minimum(a1, g))
            cum = plsc.cumsum(m.astype(jnp.int32))
            pos = jnp.where(m, bcast(rt) + cum - one_i, lsac_i)
            plsc.store_scatter(list2_v, [pos], lv)
            return rt + jnp.sum(m.astype(jnp.int32)), n1, a2

        rtop, a1, a2 = lax.fori_loop(0, nc_chunks, rp_body,
                                     (rtop, ninf, ninf))
        m1s = jnp.max(a1)
        fiv = bcast(jnp.min(jnp.where(a1 == bcast(m1s), iota, big_i)))
        first = iota == fiv
        sm = jnp.max(jnp.where(first, ninf, a1))
        m2c = jnp.max(jnp.where(first, a2, ninf))
        m2s = jnp.maximum(sm, m2c)
        subm1 = jnp.where(lane_is_sb, bcast(m1s), subm1)
        subm2 = jnp.where(lane_is_sb, bcast(m2s), subm2)
        endv = jnp.where(lane_is_sb, bcast(rtop), endv)
    sbstart_v[...] = startv
    sbend_v[...] = endv

    def p2_body(i, qoff):
        tv = t_v[pl.ds(i * L, L)]
        qm = (tv >= lovv) & (tv < hivv)
        cum = plsc.cumsum(qm.astype(jnp.int32))
        pos = jnp.where(qm, bcast(qoff) + cum - one_i, qsac_i)
        plsc.store_scatter(qval_v, [pos], tv)
        plsc.store_scatter(qidx_v, [pos], iota + bcast(i * L))
        return qoff + jnp.sum(qm.astype(jnp.int32))

    qcnt = lax.fori_loop(0, S // L, p2_body, jnp.int32(0))

    w0v = w0_v[...]
    w1v = w1_v[...]
    lcapv = jnp.full((L,), LCAP - 1, jnp.int32)

    def q_body(q, carry):
        tqv = plsc.load_gather(qval_v, [bcast(q)])
        qsbv = sub_bucket(tqv)
        rsv = plsc.load_gather(sbstart_v, [qsbv])
        rev = plsc.load_gather(sbend_v, [qsbv])
        nch = (jnp.max(rev - rsv) + L - 1) // L

        def c_body(j, ac):
            a1, a2 = ac
            idx = rsv + bcast(j * L) + iota
            valid = idx < rev
            lv = plsc.load_gather(list2_v, [jnp.minimum(idx, lcapv)])
            ok = valid & (lv <= tqv)
            g = jnp.where(ok, lv, ninf)
            n1 = jnp.maximum(a1, g)
            a2 = jnp.maximum(a2, jnp.minimum(a1, g))
            return n1, a2

        a1, a2 = lax.fori_loop(0, nch, c_body, (ninf, ninf))
        below_sb = iota < qsbv
        b1 = jnp.where(below_sb, subm1, ninf)
        b2 = jnp.where(below_sb, subm2, ninf)
        c1 = jnp.maximum(a1, b1)
        c2 = jnp.maximum(jnp.minimum(a1, b1), jnp.maximum(a2, b2))
        d1 = jnp.maximum(c1, p1)
        d2 = jnp.maximum(jnp.minimum(c1, p1), jnp.maximum(c2, p2))
        m1v = bcast(jnp.max(d1))
        fiv = bcast(jnp.min(jnp.where(d1 == m1v, iota, big_i)))
        first = iota == fiv
        sm = jnp.max(jnp.where(first, ninf, d1))
        m2c = jnp.max(jnp.where(first, d2, ninf))
        m2v = bcast(jnp.maximum(sm, m2c))
        hvv = w0v * jnp.exp(m1v - tqv) + w1v * jnp.exp(m2v - tqv)
        plsc.store_scatter(hvout_v, [jnp.where(iota == zero_i, bcast(q), qsac_i)],
                           hvv)
        return carry

    lax.fori_loop(0, qcnt, q_body, jnp.int32(0))

    qcntv = bcast(qcnt)

    def s_body(j, carry):
        idxv = qidx_v[pl.ds(j * L, L)]
        valid = (iota + bcast(j * L)) < qcntv
        idxv = jnp.where(valid, idxv, jnp.full((L,), S, jnp.int32))
        pltpu.async_copy(hvout_v.at[pl.ds(j * L, L)], hv_hbm.at[idxv],
                         sem).wait()
        return carry

    lax.fori_loop(0, (qcnt + L - 1) // L, s_body, jnp.int32(0))


_sc_kernel = functools.partial(
    pl.kernel,
    out_type=jax.ShapeDtypeStruct((S + L,), jnp.float32),
    mesh=plsc.VectorSubcoreMesh(core_axis_name="c", subcore_axis_name="s"),
    scratch_types=[
        pltpu.VMEM((H,), jnp.float32),
        pltpu.VMEM((S,), jnp.float32),
        pltpu.VMEM((L,), jnp.float32),
        pltpu.VMEM((L,), jnp.float32),
        pltpu.VMEM((LCAP,), jnp.float32),
        pltpu.VMEM((LCAP,), jnp.float32),
        pltpu.VMEM((S + L,), jnp.float32),
        pltpu.VMEM((S + L,), jnp.int32),
        pltpu.VMEM((S + L,), jnp.float32),
        pltpu.VMEM((L,), jnp.int32),
        pltpu.VMEM((L,), jnp.int32),
        pltpu.SemaphoreType.DMA,
    ],
    compiler_params=pltpu.CompilerParams(needs_layout_passes=False),
)(_sc_body)


def _combine_body(hv_ref, x_ref, out_ref):
    out_ref[...] = hv_ref[...] - x_ref[...]


@jax.jit
def kernel(x, t, history, W_hist):
    B = x.shape[0]
    t_row = t[0, :, 0]
    w0 = jnp.full((L,), W_hist[0, 0], jnp.float32)
    w1 = jnp.full((L,), W_hist[0, 1], jnp.float32)
    hv = _sc_kernel(history, t_row, w0, w1)
    hv2d = hv[:S].reshape(1, S)
    out = pl.pallas_call(
        _combine_body,
        out_shape=jax.ShapeDtypeStruct((B, S), jnp.float32),
    )(hv2d, x)
    return out
